# Initial kernel scaffold; baseline (speedup 1.0000x reference)
import jax
import jax.numpy as jnp
from jax.experimental import pallas as pl


def kernel(position_ids, pos_embed_t, pos_embed_h, pos_embed_w):
    raise NotImplementedError("write your pallas kernel here")


# speedup vs baseline: 4.1969x; 4.1969x over previous
import functools

import jax
import jax.numpy as jnp
from jax import lax
from jax.experimental import pallas as pl
from jax.experimental.pallas import tpu as pltpu
from jax.experimental.pallas import tpu_sc as plsc

N_TOKENS = 65536
DT, DH, DW = 344, 340, 340
DOUT = DT + DH + DW
HALF = 512
SPLIT_H = HALF - DT


def _pos_embed_kernel(th_hbm, hw_hbm, tab_th, tab_hw, out_hbm,
                      idx_th, idx_hw, rows, sem,
                      *, bpw, chunk):
    nc = 2
    wid = lax.axis_index("s") * nc + lax.axis_index("c")
    base = wid * bpw

    def body(i, carry):
        off = base + i * chunk
        pltpu.sync_copy(th_hbm.at[pl.ds(off, chunk)], idx_th)
        pltpu.sync_copy(hw_hbm.at[pl.ds(off, chunk)], idx_hw)
        c0 = pltpu.async_copy(tab_th.at[idx_th], rows.at[:, pl.ds(0, HALF)], sem)
        c1 = pltpu.async_copy(tab_hw.at[idx_hw], rows.at[:, pl.ds(HALF, HALF)], sem)
        c0.wait()
        c1.wait()
        pltpu.sync_copy(rows, out_hbm.at[pl.ds(off, chunk)])
        return carry

    lax.fori_loop(0, bpw // chunk, body, 0)


def kernel(position_ids, pos_embed_t, pos_embed_h, pos_embed_w):
    pid = position_ids.reshape(3, -1).astype(jnp.int32)
    th_ids = pid[0] * 64 + pid[1]
    hw_ids = pid[1] * 64 + pid[2]
    tab_th = jnp.concatenate(
        [jnp.repeat(pos_embed_t, 64, axis=0),
         jnp.tile(pos_embed_h[:, :SPLIT_H], (64, 1))], axis=-1)
    tab_hw = jnp.concatenate(
        [jnp.repeat(pos_embed_h[:, SPLIT_H:], 64, axis=0),
         jnp.tile(pos_embed_w, (64, 1))], axis=-1)

    nw = 32
    bpw = N_TOKENS // nw
    chunk = 64

    mesh = plsc.VectorSubcoreMesh(core_axis_name="c", subcore_axis_name="s")
    run = functools.partial(
        pl.kernel,
        mesh=mesh,
        out_type=jax.ShapeDtypeStruct((N_TOKENS, DOUT), jnp.float32),
        scratch_types=[
            pltpu.VMEM((chunk,), jnp.int32),
            pltpu.VMEM((chunk,), jnp.int32),
            pltpu.VMEM((chunk, DOUT), jnp.float32),
            pltpu.SemaphoreType.DMA,
        ],
    )(functools.partial(_pos_embed_kernel, bpw=bpw, chunk=chunk))
    return run(th_ids, hw_ids, tab_th, tab_hw)


# speedup vs baseline: 4.9159x; 1.1713x over previous
import functools

import jax
import jax.numpy as jnp
from jax import lax
from jax.experimental import pallas as pl
from jax.experimental.pallas import tpu as pltpu
from jax.experimental.pallas import tpu_sc as plsc

N_TOKENS = 65536
DT, DH, DW = 344, 340, 340
DOUT = DT + DH + DW
HALF = 512
SPLIT_H = HALF - DT


def _pos_embed_kernel(th_hbm, hw_hbm, tab_th, tab_hw, out_hbm,
                      idx_th, idx_hw, rows0, rows1, g0, g1, w0, w1,
                      *, bpw, chunk):
    nc = 2
    wid = lax.axis_index("s") * nc + lax.axis_index("c")
    base = wid * bpw
    nsteps = bpw // chunk

    rows = (rows0, rows1)
    gsem = (g0, g1)
    wsem = (w0, w1)

    pltpu.sync_copy(th_hbm.at[pl.ds(base, bpw)], idx_th)
    pltpu.sync_copy(hw_hbm.at[pl.ds(base, bpw)], idx_hw)

    def gather(i, b):
        sl = pl.ds(i * chunk, chunk)
        pltpu.async_copy(tab_th.at[idx_th.at[sl]],
                         rows[b].at[:, pl.ds(0, HALF)], gsem[b])
        pltpu.async_copy(tab_hw.at[idx_hw.at[sl]],
                         rows[b].at[:, pl.ds(HALF, HALF)], gsem[b])

    def wait_gather(b):
        pltpu.make_async_copy(rows[b], out_hbm.at[pl.ds(0, chunk)],
                              gsem[b]).wait()

    def write(i, b):
        pltpu.async_copy(rows[b], out_hbm.at[pl.ds(base + i * chunk, chunk)],
                         wsem[b])

    def wait_write(b):
        pltpu.make_async_copy(rows[b], out_hbm.at[pl.ds(0, chunk)],
                              wsem[b]).wait()

    gather(0, 0)

    def body(i, carry):
        for b in (0, 1):
            @pl.when(i % 2 == b)
            def _do():
                @pl.when(i >= 2)
                def _drain():
                    wait_write(b)
                gather(i, b)
                wait_gather(1 - b)
                write(i - 1, 1 - b)
        return carry

    lax.fori_loop(1, nsteps, body, 0)
    last = nsteps - 1
    wait_gather(last % 2)
    write(last, last % 2)
    wait_write(0)
    wait_write(1)


def kernel(position_ids, pos_embed_t, pos_embed_h, pos_embed_w):
    pid = position_ids.reshape(3, -1).astype(jnp.int32)
    th_ids = pid[0] * 64 + pid[1]
    hw_ids = pid[1] * 64 + pid[2]
    tab_th = jnp.concatenate(
        [jnp.repeat(pos_embed_t, 64, axis=0),
         jnp.tile(pos_embed_h[:, :SPLIT_H], (64, 1))], axis=-1)
    tab_hw = jnp.concatenate(
        [jnp.repeat(pos_embed_h[:, SPLIT_H:], 64, axis=0),
         jnp.tile(pos_embed_w, (64, 1))], axis=-1)

    nw = 32
    bpw = N_TOKENS // nw
    chunk = 32

    mesh = plsc.VectorSubcoreMesh(core_axis_name="c", subcore_axis_name="s")
    run = functools.partial(
        pl.kernel,
        mesh=mesh,
        out_type=jax.ShapeDtypeStruct((N_TOKENS, DOUT), jnp.float32),
        scratch_types=[
            pltpu.VMEM((bpw,), jnp.int32),
            pltpu.VMEM((bpw,), jnp.int32),
            pltpu.VMEM((chunk, DOUT), jnp.float32),
            pltpu.VMEM((chunk, DOUT), jnp.float32),
            pltpu.SemaphoreType.DMA,
            pltpu.SemaphoreType.DMA,
            pltpu.SemaphoreType.DMA,
            pltpu.SemaphoreType.DMA,
        ],
    )(functools.partial(_pos_embed_kernel, bpw=bpw, chunk=chunk))
    return run(th_ids, hw_ids, tab_th, tab_hw)


# speedup vs baseline: 4.9308x; 1.0030x over previous
import functools

import jax
import jax.numpy as jnp
from jax import lax
from jax.experimental import pallas as pl
from jax.experimental.pallas import tpu as pltpu
from jax.experimental.pallas import tpu_sc as plsc

N_TOKENS = 65536
DT, DH, DW = 344, 340, 340
DOUT = DT + DH + DW
HALF = 512
SPLIT_H = HALF - DT


def _pos_embed_kernel(th_hbm, hw_hbm, tab_th, tab_hw, out_hbm,
                      idx_th, idx_hw, *bufs_and_sems, bpw, chunk, nbuf):
    rows = bufs_and_sems[:nbuf]
    gsem = bufs_and_sems[nbuf:2 * nbuf]
    wsem = bufs_and_sems[2 * nbuf:3 * nbuf]
    nc = 2
    wid = lax.axis_index("s") * nc + lax.axis_index("c")
    base = wid * bpw
    nsteps = bpw // chunk

    pltpu.sync_copy(th_hbm.at[pl.ds(base, bpw)], idx_th)
    pltpu.sync_copy(hw_hbm.at[pl.ds(base, bpw)], idx_hw)

    def gather(i, b):
        sl = pl.ds(i * chunk, chunk)
        pltpu.async_copy(tab_th.at[idx_th.at[sl]],
                         rows[b].at[:, pl.ds(0, HALF)], gsem[b])
        pltpu.async_copy(tab_hw.at[idx_hw.at[sl]],
                         rows[b].at[:, pl.ds(HALF, HALF)], gsem[b])

    def wait_gather(b):
        pltpu.make_async_copy(rows[b], out_hbm.at[pl.ds(0, chunk)],
                              gsem[b]).wait()

    def write(i, b):
        pltpu.async_copy(rows[b], out_hbm.at[pl.ds(base + i * chunk, chunk)],
                         wsem[b])

    def wait_write(b):
        pltpu.make_async_copy(rows[b], out_hbm.at[pl.ds(0, chunk)],
                              wsem[b]).wait()

    for b in range(nbuf - 1):
        gather(b, b)

    def body(i, carry):
        for b in range(nbuf):
            @pl.when((i + nbuf - 1) % nbuf == b)
            def _issue():
                @pl.when(i >= 1)
                def _drain():
                    wait_write(b)
                gather(i + nbuf - 1, b)
            @pl.when(i % nbuf == b)
            def _consume():
                wait_gather(b)
                write(i, b)
        return carry

    lax.fori_loop(0, nsteps - (nbuf - 1), body, 0)
    for j in range(nsteps - (nbuf - 1), nsteps):
        b = j % nbuf
        wait_gather(b)
        write(j, b)
    for b in range(nbuf):
        wait_write(b)


def kernel(position_ids, pos_embed_t, pos_embed_h, pos_embed_w):
    pid = position_ids.reshape(3, -1).astype(jnp.int32)
    th_ids = pid[0] * 64 + pid[1]
    hw_ids = pid[1] * 64 + pid[2]
    tab_th = jnp.concatenate(
        [jnp.repeat(pos_embed_t, 64, axis=0),
         jnp.tile(pos_embed_h[:, :SPLIT_H], (64, 1))], axis=-1)
    tab_hw = jnp.concatenate(
        [jnp.repeat(pos_embed_h[:, SPLIT_H:], 64, axis=0),
         jnp.tile(pos_embed_w, (64, 1))], axis=-1)

    nw = 32
    bpw = N_TOKENS // nw
    chunk = 16
    nbuf = 4

    mesh = plsc.VectorSubcoreMesh(core_axis_name="c", subcore_axis_name="s")
    run = functools.partial(
        pl.kernel,
        mesh=mesh,
        out_type=jax.ShapeDtypeStruct((N_TOKENS, DOUT), jnp.float32),
        scratch_types=(
            [pltpu.VMEM((bpw,), jnp.int32)] * 2
            + [pltpu.VMEM((chunk, DOUT), jnp.float32)] * nbuf
            + [pltpu.SemaphoreType.DMA] * (2 * nbuf)
        ),
    )(functools.partial(_pos_embed_kernel, bpw=bpw, chunk=chunk, nbuf=nbuf))
    return run(th_ids, hw_ids, tab_th, tab_hw)
